# baseline (speedup 1.0000x reference)
import functools
import math

import jax
import jax.numpy as jnp
from jax import lax
from jax.experimental import pallas as pl
from jax.experimental.pallas import tpu as pltpu
from jax.experimental.pallas import tpu_sc as plsc

_LANES = 16
_NUM_WORKERS = 32
_NBUF = 3
_CHUNK = 32


def kernel(input_ids, token_table, pos_table):
    B, S = input_ids.shape
    V, D = token_table.shape
    N = B * S
    scale = math.sqrt(float(D))
    s_per_w = S // _NUM_WORKERS
    nvec = D // _LANES
    sub = s_per_w // _CHUNK
    nch = B * sub

    mesh = plsc.VectorSubcoreMesh(core_axis_name="c", subcore_axis_name="s")

    @functools.partial(
        pl.kernel,
        mesh=mesh,
        out_type=jax.ShapeDtypeStruct((N, D), jnp.float32),
        scratch_types=[
            pltpu.VMEM((B, s_per_w), jnp.int32),
            pltpu.VMEM((s_per_w, D), jnp.float32),
            pltpu.VMEM((_CHUNK, D), jnp.float32),
            pltpu.VMEM((_CHUNK, D), jnp.float32),
            pltpu.VMEM((_CHUNK, D), jnp.float32),
            pltpu.SemaphoreType.DMA,
            pltpu.SemaphoreType.DMA,
            pltpu.SemaphoreType.DMA,
            pltpu.SemaphoreType.DMA,
            pltpu.SemaphoreType.DMA,
            pltpu.SemaphoreType.DMA,
            pltpu.SemaphoreType.DMA,
            pltpu.SemaphoreType.DMA,
        ],
    )
    def body(ids_hbm, tok_hbm, pos_hbm, out_hbm, idx_v, pos_v, t0, t1, t2,
             g0, g1, g2, o0, o1, o2, isem, psem):
        wid = lax.axis_index("s") * 2 + lax.axis_index("c")
        s0 = wid * s_per_w
        idx_cps = [
            pltpu.async_copy(ids_hbm.at[pl.ds(b * S + s0, s_per_w)],
                             idx_v.at[b], isem)
            for b in range(B)
        ]
        pos_cp = pltpu.async_copy(pos_hbm.at[pl.ds(s0, s_per_w)], pos_v, psem)
        for cp in idx_cps:
            cp.wait()

        tbufs = [t0, t1, t2]
        gsems = [g0, g1, g2]
        osems = [o0, o1, o2]
        gathers = [None] * _NBUF
        stores = [None] * _NBUF

        def start_gather(j):
            b, r = divmod(j, sub)
            slot = j % _NBUF
            gathers[slot] = pltpu.async_copy(
                tok_hbm.at[idx_v.at[b, pl.ds(r * _CHUNK, _CHUNK)]],
                tbufs[slot], gsems[slot])

        start_gather(0)
        start_gather(1)
        pos_cp.wait()
        for j in range(nch):
            slot = j % _NBUF
            if j + 2 < nch:
                nslot = (j + 2) % _NBUF
                if stores[nslot] is not None:
                    stores[nslot].wait()
                start_gather(j + 2)
            gathers[slot].wait()
            buf = tbufs[slot]
            b, r = divmod(j, sub)
            poff = r * _CHUNK

            def rows(i, _, buf=buf, poff=poff):
                for u in range(2):
                    for k in range(nvec):
                        sl = pl.ds(k * _LANES, _LANES)
                        buf[i * 2 + u, sl] = (
                            buf[i * 2 + u, sl] * scale
                            + pos_v[poff + i * 2 + u, sl])
                return 0

            lax.fori_loop(0, _CHUNK // 2, rows, 0)
            stores[slot] = pltpu.async_copy(
                buf, out_hbm.at[pl.ds(b * S + s0 + poff, _CHUNK)],
                osems[slot])
        for st in stores:
            st.wait()

    out = body(input_ids.reshape(N), token_table, pos_table)
    return out.reshape(B, S, D)


# speedup vs baseline: 1.4804x; 1.4804x over previous
import functools
import math

import jax
import jax.numpy as jnp
from jax import lax
from jax.experimental import pallas as pl
from jax.experimental.pallas import tpu as pltpu
from jax.experimental.pallas import tpu_sc as plsc

_LANES = 16
_NUM_WORKERS = 32


def kernel(input_ids, token_table, pos_table):
    B, S = input_ids.shape
    V, D = token_table.shape
    N = B * S
    scale = math.sqrt(float(D))
    s_per_w = S // _NUM_WORKERS
    nvec = D // _LANES

    mesh = plsc.VectorSubcoreMesh(core_axis_name="c", subcore_axis_name="s")

    @functools.partial(
        pl.kernel,
        mesh=mesh,
        out_type=jax.ShapeDtypeStruct((N, D), jnp.float32),
        scratch_types=[
            pltpu.VMEM((B, s_per_w), jnp.int32),
            pltpu.VMEM((s_per_w, D), jnp.float32),
            pltpu.VMEM((s_per_w, D), jnp.float32),
            pltpu.VMEM((s_per_w, D), jnp.float32),
            pltpu.SemaphoreType.DMA,
            pltpu.SemaphoreType.DMA,
            pltpu.SemaphoreType.DMA,
            pltpu.SemaphoreType.DMA,
            pltpu.SemaphoreType.DMA,
            pltpu.SemaphoreType.DMA,
        ],
    )
    def body(ids_hbm, tok_hbm, pos_hbm, out_hbm, idx_v, pos_v, t0, t1,
             g0, g1, o0, o1, isem, psem):
        wid = lax.axis_index("s") * 2 + lax.axis_index("c")
        s0 = wid * s_per_w
        idx_cps = [
            pltpu.async_copy(ids_hbm.at[pl.ds(b * S + s0, s_per_w)],
                             idx_v.at[b], isem)
            for b in range(B)
        ]
        pos_cp = pltpu.async_copy(pos_hbm.at[pl.ds(s0, s_per_w)], pos_v, psem)
        for cp in idx_cps:
            cp.wait()

        tbufs = [t0, t1]
        gsems = [g0, g1]
        osems = [o0, o1]
        gathers = [None, None]
        stores = [None, None]
        gathers[0] = pltpu.async_copy(tok_hbm.at[idx_v.at[0]], t0, g0)
        pos_cp.wait()
        for b in range(B):
            cur = b % 2
            nxt = (b + 1) % 2
            if b + 1 < B:
                if stores[nxt] is not None:
                    stores[nxt].wait()
                gathers[nxt] = pltpu.async_copy(
                    tok_hbm.at[idx_v.at[b + 1]], tbufs[nxt], gsems[nxt])
            gathers[cur].wait()
            buf = tbufs[cur]

            def row(r, _, buf=buf):
                for k in range(nvec):
                    sl = pl.ds(k * _LANES, _LANES)
                    buf[r, sl] = buf[r, sl] * scale + pos_v[r, sl]
                return 0

            lax.fori_loop(0, s_per_w, row, 0)
            stores[cur] = pltpu.async_copy(
                buf, out_hbm.at[pl.ds(b * S + s0, s_per_w)], osems[cur])
        stores[0].wait()
        stores[1].wait()

    out = body(input_ids.reshape(N), token_table, pos_table)
    return out.reshape(B, S, D)
